# Initial kernel scaffold; baseline (speedup 1.0000x reference)
import jax
import jax.numpy as jnp
from jax.experimental import pallas as pl


def kernel(hidden_states, top_k_index, top_k_weights, gate_up_proj, down_proj):
    raise NotImplementedError("write your pallas kernel here")


# speedup vs baseline: 1.2264x; 1.2264x over previous
import functools

import jax
import jax.numpy as jnp
from jax import lax
from jax.experimental import pallas as pl
from jax.experimental.pallas import tpu as pltpu
from jax.experimental.pallas import tpu_sc as plsc

E = 8
K = 2
T = 2048
H = 1024
F = 4096
A = T * K
BLK = 256
NB = 24
APAD = NB * BLK
NJ = 2
CH = F // NJ

NW = 32

GROWS = APAD // NW
GCH = 96
CROWS = T // NW
CCH = 32


@functools.cache
def _sc_mesh():
    return plsc.VectorSubcoreMesh(core_axis_name="c", subcore_axis_name="s")


@functools.cache
def _sc_gather_call():
    @functools.partial(
        pl.kernel,
        out_type=jax.ShapeDtypeStruct((APAD, H), jnp.float32),
        mesh=_sc_mesh(),
        scratch_types=[
            pltpu.VMEM((GCH,), jnp.int32),
            pltpu.VMEM((GCH, H), jnp.float32),
            pltpu.SemaphoreType.DMA,
        ],
    )
    def _sc_gather(tok_hbm, hid_hbm, out_hbm, idx_v, rows_v, sem):
        wid = lax.axis_index("s") * 2 + lax.axis_index("c")
        for c in range(GROWS // GCH):
            base = wid * GROWS + c * GCH
            pltpu.sync_copy(tok_hbm.at[pl.ds(base, GCH)], idx_v)
            pltpu.async_copy(hid_hbm.at[idx_v], rows_v, sem).wait()
            pltpu.sync_copy(rows_v, out_hbm.at[pl.ds(base, GCH)])

    return _sc_gather


@functools.cache
def _sc_combine_call():
    @functools.partial(
        pl.kernel,
        out_type=jax.ShapeDtypeStruct((T, H), jnp.float32),
        mesh=_sc_mesh(),
        scratch_types=[
            pltpu.VMEM((CCH,), jnp.int32),
            pltpu.VMEM((CCH,), jnp.int32),
            pltpu.VMEM((CCH, H), jnp.float32),
            pltpu.VMEM((CCH, H), jnp.float32),
            pltpu.SemaphoreType.DMA,
        ],
    )
    def _sc_combine(p0_hbm, p1_hbm, y_hbm, out_hbm, i0_v, i1_v, r0_v, r1_v, sem):
        wid = lax.axis_index("s") * 2 + lax.axis_index("c")
        for c in range(CROWS // CCH):
            base = wid * CROWS + c * CCH
            pltpu.sync_copy(p0_hbm.at[pl.ds(base, CCH)], i0_v)
            pltpu.sync_copy(p1_hbm.at[pl.ds(base, CCH)], i1_v)
            cp0 = pltpu.async_copy(y_hbm.at[i0_v], r0_v, sem)
            cp1 = pltpu.async_copy(y_hbm.at[i1_v], r1_v, sem)
            cp0.wait()
            cp1.wait()

            def _add16(i, _):
                r = i // (H // 16)
                k = (i % (H // 16)) * 16
                r0_v[r, pl.ds(k, 16)] = r0_v[r, pl.ds(k, 16)] + r1_v[r, pl.ds(k, 16)]
                return 0

            lax.fori_loop(0, CCH * (H // 16), _add16, 0)
            pltpu.sync_copy(r0_v, out_hbm.at[pl.ds(base, CCH)])

    return _sc_combine


def _ffn_body(be_ref, x_ref, g_ref, u_ref, d_ref, w_ref, o_ref):
    del be_ref
    j = pl.program_id(1)
    x = x_ref[...]
    g = lax.dot_general(x, g_ref[0], (((1,), (1,)), ((), ())),
                        preferred_element_type=jnp.float32)
    u = lax.dot_general(x, u_ref[0], (((1,), (1,)), ((), ())),
                        preferred_element_type=jnp.float32)
    a = jax.nn.gelu(g, approximate=True) * u
    a = a * w_ref[0, 0, :][:, None]
    p = lax.dot_general(a, d_ref[0], (((1,), (1,)), ((), ())),
                        preferred_element_type=jnp.float32)

    @pl.when(j == 0)
    def _():
        o_ref[...] = p

    @pl.when(j != 0)
    def _():
        o_ref[...] += p


_FFN_GRID = pltpu.PrefetchScalarGridSpec(
    num_scalar_prefetch=1,
    grid=(NB, NJ),
    in_specs=[
        pl.BlockSpec((BLK, H), lambda b, j, be: (b, 0)),
        pl.BlockSpec((1, CH, H), lambda b, j, be: (be[b], j, 0)),
        pl.BlockSpec((1, CH, H), lambda b, j, be: (be[b], NJ + j, 0)),
        pl.BlockSpec((1, H, CH), lambda b, j, be: (be[b], 0, j)),
        pl.BlockSpec((1, 1, BLK), lambda b, j, be: (b, 0, 0)),
    ],
    out_specs=pl.BlockSpec((BLK, H), lambda b, j, be: (b, 0)),
)

_ffn_call = pl.pallas_call(
    _ffn_body,
    grid_spec=_FFN_GRID,
    out_shape=jax.ShapeDtypeStruct((APAD, H), jnp.float32),
    compiler_params=pltpu.CompilerParams(
        dimension_semantics=("arbitrary", "arbitrary"),
    ),
)


def kernel(hidden_states, top_k_index, top_k_weights, gate_up_proj, down_proj):
    e_flat = top_k_index.reshape(A).astype(jnp.int32)
    w_flat = top_k_weights.reshape(A).astype(jnp.float32)

    oh = (e_flat[:, None] == jnp.arange(E, dtype=jnp.int32)[None, :]).astype(jnp.int32)
    csum = jnp.cumsum(oh, axis=0)
    counts = csum[-1]
    bpe = (counts + BLK - 1) // BLK
    cb = jnp.cumsum(bpe)
    bstart = (cb - bpe) * BLK
    rank = jnp.take_along_axis(csum, e_flat[:, None], axis=1)[:, 0] - 1
    pos = (bstart[e_flat] + rank).astype(jnp.int32)

    tok = (jnp.arange(A, dtype=jnp.int32) // K)
    row_token = jnp.zeros((APAD,), jnp.int32).at[pos].set(tok)
    row_weight = jnp.zeros((APAD,), jnp.float32).at[pos].set(w_flat)
    block_expert = jnp.minimum(
        jnp.sum((jnp.arange(NB, dtype=jnp.int32)[:, None] >= cb[None, :]), axis=1),
        E - 1,
    ).astype(jnp.int32)
    pos01 = pos.reshape(T, K)

    x_sorted = _sc_gather_call()(row_token, hidden_states)
    y_sorted = _ffn_call(
        block_expert,
        x_sorted,
        gate_up_proj,
        gate_up_proj,
        down_proj,
        row_weight.reshape(NB, 1, BLK),
    )
    final = _sc_combine_call()(pos01[:, 0], pos01[:, 1], y_sorted)
    return final


# speedup vs baseline: 1.3784x; 1.1240x over previous
import functools

import jax
import jax.numpy as jnp
from jax import lax
from jax.experimental import pallas as pl
from jax.experimental.pallas import tpu as pltpu
from jax.experimental.pallas import tpu_sc as plsc

E = 8
K = 2
T = 2048
H = 1024
F = 4096
A = T * K
BLK = 256
NB = 24
APAD = NB * BLK
NJ = 4
CH = F // NJ

NW = 32

GROWS = APAD // NW
GCH = 32
GNCH = GROWS // GCH
GNBUF = 3
CROWS = T // NW
CCH = 32


@functools.cache
def _sc_mesh():
    return plsc.VectorSubcoreMesh(core_axis_name="c", subcore_axis_name="s")


@functools.cache
def _sc_gather_call():
    @functools.partial(
        pl.kernel,
        out_type=jax.ShapeDtypeStruct((APAD, H), jnp.float32),
        mesh=_sc_mesh(),
        scratch_types=[
            pltpu.VMEM((GNCH, GCH), jnp.int32),
            *[pltpu.VMEM((GCH, H), jnp.float32) for _ in range(GNBUF)],
            pltpu.SemaphoreType.DMA,
            pltpu.SemaphoreType.DMA,
        ],
    )
    def _sc_gather(tok_hbm, hid_hbm, out_hbm, idx_v, b0, b1, b2, gsem, wsem):
        bufs = (b0, b1, b2)
        wid = lax.axis_index("s") * 2 + lax.axis_index("c")
        base = wid * GROWS
        pltpu.sync_copy(tok_hbm.at[wid], idx_v)
        gds = [None] * GNCH
        wbs = [None] * GNCH
        for c in range(GNBUF):
            gds[c] = pltpu.async_copy(hid_hbm.at[idx_v.at[c]], bufs[c], gsem)
        for c in range(GNCH):
            gds[c].wait()
            wbs[c] = pltpu.async_copy(
                bufs[c % GNBUF], out_hbm.at[pl.ds(base + c * GCH, GCH)], wsem)
            nxt = c + GNBUF
            if nxt < GNCH:
                wbs[c].wait()
                gds[nxt] = pltpu.async_copy(
                    hid_hbm.at[idx_v.at[nxt]], bufs[nxt % GNBUF], gsem)
        for c in range(GNCH - GNBUF, GNCH):
            wbs[c].wait()

    return _sc_gather


@functools.cache
def _sc_combine_call():
    @functools.partial(
        pl.kernel,
        out_type=jax.ShapeDtypeStruct((T, H), jnp.float32),
        mesh=_sc_mesh(),
        scratch_types=[
            pltpu.VMEM((CCH,), jnp.int32),
            pltpu.VMEM((CCH,), jnp.int32),
            pltpu.VMEM((CCH, H), jnp.float32),
            pltpu.VMEM((CCH, H), jnp.float32),
            pltpu.SemaphoreType.DMA,
        ],
    )
    def _sc_combine(p0_hbm, p1_hbm, y_hbm, out_hbm, i0_v, i1_v, r0_v, r1_v, sem):
        wid = lax.axis_index("s") * 2 + lax.axis_index("c")
        for c in range(CROWS // CCH):
            base = wid * CROWS + c * CCH
            pltpu.sync_copy(p0_hbm.at[pl.ds(base, CCH)], i0_v)
            pltpu.sync_copy(p1_hbm.at[pl.ds(base, CCH)], i1_v)
            cp0 = pltpu.async_copy(y_hbm.at[i0_v], r0_v, sem)
            cp1 = pltpu.async_copy(y_hbm.at[i1_v], r1_v, sem)
            cp0.wait()
            cp1.wait()

            def _add16(i, _):
                r = i // (H // 16)
                k = (i % (H // 16)) * 16
                r0_v[r, pl.ds(k, 16)] = r0_v[r, pl.ds(k, 16)] + r1_v[r, pl.ds(k, 16)]
                return 0

            lax.fori_loop(0, CCH * (H // 16), _add16, 0)
            pltpu.sync_copy(r0_v, out_hbm.at[pl.ds(base, CCH)])

    return _sc_combine


def _ffn_body(be_ref, x_ref, g_ref, u_ref, d_ref, w_ref, o_ref):
    del be_ref
    j = pl.program_id(0)
    b = pl.program_id(1)
    x = x_ref[...]
    g = lax.dot_general(x, g_ref[0], (((1,), (1,)), ((), ())),
                        preferred_element_type=jnp.float32)
    u = lax.dot_general(x, u_ref[0], (((1,), (1,)), ((), ())),
                        preferred_element_type=jnp.float32)
    a = jax.nn.gelu(g, approximate=True) * u
    a = a * w_ref[0, 0, :][:, None]
    p = lax.dot_general(a, d_ref[0], (((1,), (1,)), ((), ())),
                        preferred_element_type=jnp.float32)

    @pl.when(j == 0)
    def _():
        o_ref[pl.ds(b * BLK, BLK), :] = p

    @pl.when(j != 0)
    def _():
        o_ref[pl.ds(b * BLK, BLK), :] += p


_FFN_GRID = pltpu.PrefetchScalarGridSpec(
    num_scalar_prefetch=1,
    grid=(NJ, NB),
    in_specs=[
        pl.BlockSpec((BLK, H), lambda j, b, be: (b, 0)),
        pl.BlockSpec((1, CH, H), lambda j, b, be: (be[b], j, 0)),
        pl.BlockSpec((1, CH, H), lambda j, b, be: (be[b], NJ + j, 0)),
        pl.BlockSpec((1, H, CH), lambda j, b, be: (be[b], 0, j)),
        pl.BlockSpec((1, 1, BLK), lambda j, b, be: (b, 0, 0)),
    ],
    out_specs=pl.BlockSpec((APAD, H), lambda j, b, be: (0, 0)),
)

_ffn_call = pl.pallas_call(
    _ffn_body,
    grid_spec=_FFN_GRID,
    out_shape=jax.ShapeDtypeStruct((APAD, H), jnp.float32),
    compiler_params=pltpu.CompilerParams(
        dimension_semantics=("arbitrary", "arbitrary"),
    ),
)


def kernel(hidden_states, top_k_index, top_k_weights, gate_up_proj, down_proj):
    e_flat = top_k_index.reshape(A).astype(jnp.int32)
    w_flat = top_k_weights.reshape(A).astype(jnp.float32)

    oh = (e_flat[:, None] == jnp.arange(E, dtype=jnp.int32)[None, :]).astype(jnp.int32)
    csum = jnp.cumsum(oh, axis=0)
    counts = csum[-1]
    bpe = (counts + BLK - 1) // BLK
    cb = jnp.cumsum(bpe)
    bstart = (cb - bpe) * BLK
    rank = jnp.take_along_axis(csum, e_flat[:, None], axis=1)[:, 0] - 1
    pos = (bstart[e_flat] + rank).astype(jnp.int32)

    tok = (jnp.arange(A, dtype=jnp.int32) // K)
    row_token = jnp.zeros((APAD,), jnp.int32).at[pos].set(tok)
    row_weight = jnp.zeros((APAD,), jnp.float32).at[pos].set(w_flat)
    block_expert = jnp.minimum(
        jnp.sum((jnp.arange(NB, dtype=jnp.int32)[:, None] >= cb[None, :]), axis=1),
        E - 1,
    ).astype(jnp.int32)
    pos01 = pos.reshape(T, K)

    x_sorted = _sc_gather_call()(row_token.reshape(NW, GNCH, GCH), hidden_states)
    y_sorted = _ffn_call(
        block_expert,
        x_sorted,
        gate_up_proj,
        gate_up_proj,
        down_proj,
        row_weight.reshape(NB, 1, BLK),
    )
    final = _sc_combine_call()(pos01[:, 0], pos01[:, 1], y_sorted)
    return final


# speedup vs baseline: 1.7805x; 1.2917x over previous
import functools

import jax
import jax.numpy as jnp
from jax import lax
from jax.experimental import pallas as pl
from jax.experimental.pallas import tpu as pltpu
from jax.experimental.pallas import tpu_sc as plsc

E = 8
K = 2
T = 2048
H = 1024
F = 4096
A = T * K
BLK = 256
NB = 24
APAD = NB * BLK
NJ = 4
CH = F // NJ

NW = 32

GROWS = APAD // NW
GCH = 32
GNCH = GROWS // GCH
GNBUF = 3
CROWS = T // NW
CCH = 32


@functools.cache
def _sc_mesh():
    return plsc.VectorSubcoreMesh(core_axis_name="c", subcore_axis_name="s")


@functools.cache
def _sc_dispatch_call():
    @functools.partial(
        pl.kernel,
        out_type=jax.ShapeDtypeStruct((APAD, H), jnp.float32),
        mesh=_sc_mesh(),
        scratch_types=[
            pltpu.VMEM((K, CROWS), jnp.int32),
            pltpu.VMEM((CROWS, H), jnp.float32),
            pltpu.SemaphoreType.DMA,
        ],
    )
    def _sc_dispatch(p01_hbm, hid_hbm, out_hbm, idx_v, rows_v, ssem):
        wid = lax.axis_index("s") * 2 + lax.axis_index("c")
        base = wid * CROWS
        pltpu.sync_copy(p01_hbm.at[wid], idx_v)
        pltpu.sync_copy(hid_hbm.at[pl.ds(base, CROWS)], rows_v)
        s0 = pltpu.async_copy(rows_v, out_hbm.at[idx_v.at[0]], ssem)
        s1 = pltpu.async_copy(rows_v, out_hbm.at[idx_v.at[1]], ssem)
        s0.wait()
        s1.wait()

    return _sc_dispatch


@functools.cache
def _sc_combine_call():
    @functools.partial(
        pl.kernel,
        out_type=jax.ShapeDtypeStruct((T, H), jnp.float32),
        mesh=_sc_mesh(),
        scratch_types=[
            pltpu.VMEM((CCH,), jnp.int32),
            pltpu.VMEM((CCH,), jnp.int32),
            pltpu.VMEM((CCH, H), jnp.float32),
            pltpu.VMEM((CCH, H), jnp.float32),
            pltpu.SemaphoreType.DMA,
        ],
    )
    def _sc_combine(p0_hbm, p1_hbm, y_hbm, out_hbm, i0_v, i1_v, r0_v, r1_v, sem):
        wid = lax.axis_index("s") * 2 + lax.axis_index("c")
        for c in range(CROWS // CCH):
            base = wid * CROWS + c * CCH
            pltpu.sync_copy(p0_hbm.at[pl.ds(base, CCH)], i0_v)
            pltpu.sync_copy(p1_hbm.at[pl.ds(base, CCH)], i1_v)
            cp0 = pltpu.async_copy(y_hbm.at[i0_v], r0_v, sem)
            cp1 = pltpu.async_copy(y_hbm.at[i1_v], r1_v, sem)
            cp0.wait()
            cp1.wait()

            def _add16(i, _):
                r = i // (H // 16)
                k = (i % (H // 16)) * 16
                r0_v[r, pl.ds(k, 16)] = r0_v[r, pl.ds(k, 16)] + r1_v[r, pl.ds(k, 16)]
                return 0

            lax.fori_loop(0, CCH * (H // 16), _add16, 0)
            pltpu.sync_copy(r0_v, out_hbm.at[pl.ds(base, CCH)])

    return _sc_combine


def _ffn_body(be_ref, x_ref, g_ref, u_ref, d_ref, w_ref, o_ref):
    del be_ref
    j = pl.program_id(0)
    b = pl.program_id(1)
    x = x_ref[...]
    g = lax.dot_general(x, g_ref[0], (((1,), (1,)), ((), ())),
                        preferred_element_type=jnp.float32)
    u = lax.dot_general(x, u_ref[0], (((1,), (1,)), ((), ())),
                        preferred_element_type=jnp.float32)
    a = jax.nn.gelu(g, approximate=True) * u
    a = a * w_ref[0, 0, :][:, None]
    p = lax.dot_general(a, d_ref[0], (((1,), (1,)), ((), ())),
                        preferred_element_type=jnp.float32)

    @pl.when(j == 0)
    def _():
        o_ref[pl.ds(b * BLK, BLK), :] = p

    @pl.when(j != 0)
    def _():
        o_ref[pl.ds(b * BLK, BLK), :] += p


_FFN_GRID = pltpu.PrefetchScalarGridSpec(
    num_scalar_prefetch=1,
    grid=(NJ, NB),
    in_specs=[
        pl.BlockSpec((BLK, H), lambda j, b, be: (b, 0)),
        pl.BlockSpec((1, CH, H), lambda j, b, be: (be[b], j, 0)),
        pl.BlockSpec((1, CH, H), lambda j, b, be: (be[b], NJ + j, 0)),
        pl.BlockSpec((1, H, CH), lambda j, b, be: (be[b], 0, j)),
        pl.BlockSpec((1, 1, BLK), lambda j, b, be: (b, 0, 0)),
    ],
    out_specs=pl.BlockSpec((APAD, H), lambda j, b, be: (0, 0)),
)

_ffn_call = pl.pallas_call(
    _ffn_body,
    grid_spec=_FFN_GRID,
    out_shape=jax.ShapeDtypeStruct((APAD, H), jnp.float32),
    compiler_params=pltpu.CompilerParams(
        dimension_semantics=("arbitrary", "arbitrary"),
    ),
)


def kernel(hidden_states, top_k_index, top_k_weights, gate_up_proj, down_proj):
    e_flat = top_k_index.reshape(A).astype(jnp.int32)
    w_flat = top_k_weights.reshape(A).astype(jnp.float32)

    oh = (e_flat[:, None] == jnp.arange(E, dtype=jnp.int32)[None, :]).astype(jnp.int32)
    csum = jnp.cumsum(oh, axis=0)
    counts = csum[-1]
    bpe = (counts + BLK - 1) // BLK
    cb = jnp.cumsum(bpe)
    bstart = (cb - bpe) * BLK
    rank = jnp.take_along_axis(csum, e_flat[:, None], axis=1)[:, 0] - 1
    pos = (bstart[e_flat] + rank).astype(jnp.int32)

    row_weight = jnp.zeros((APAD,), jnp.float32).at[pos].set(w_flat)
    block_expert = jnp.minimum(
        jnp.sum((jnp.arange(NB, dtype=jnp.int32)[:, None] >= cb[None, :]), axis=1),
        E - 1,
    ).astype(jnp.int32)
    pos01 = pos.reshape(T, K)

    p01 = pos01.reshape(NW, CROWS, K).transpose(0, 2, 1)
    x_sorted = _sc_dispatch_call()(p01, hidden_states)
    y_sorted = _ffn_call(
        block_expert,
        x_sorted,
        gate_up_proj,
        gate_up_proj,
        down_proj,
        row_weight.reshape(NB, 1, BLK),
    )
    final = _sc_combine_call()(pos01[:, 0], pos01[:, 1], y_sorted)
    return final


# speedup vs baseline: 1.9395x; 1.0893x over previous
import functools

import jax
import jax.numpy as jnp
from jax import lax
from jax.experimental import pallas as pl
from jax.experimental.pallas import tpu as pltpu
from jax.experimental.pallas import tpu_sc as plsc

E = 8
K = 2
T = 2048
H = 1024
F = 4096
A = T * K
BLK = 256
NB = 24
APAD = NB * BLK
NJ = 4
CH = F // NJ

NW = 32

GROWS = APAD // NW
GCH = 32
GNCH = GROWS // GCH
GNBUF = 3
CROWS = T // NW
CCH = 32


@functools.cache
def _sc_mesh():
    return plsc.VectorSubcoreMesh(core_axis_name="c", subcore_axis_name="s")


@functools.cache
def _sc_route_call():
    @functools.partial(
        pl.kernel,
        out_type=(
            jax.ShapeDtypeStruct((A,), jnp.int32),
            jax.ShapeDtypeStruct((APAD,), jnp.float32),
        ),
        mesh=_sc_mesh(),
        scratch_types=[
            pltpu.VMEM((A,), jnp.int32),
            pltpu.VMEM((A,), jnp.float32),
            pltpu.VMEM((A,), jnp.int32),
            pltpu.VMEM((A,), jnp.int32),
            pltpu.VMEM((APAD,), jnp.float32),
            pltpu.VMEM((128,), jnp.int32),
            pltpu.VMEM((128,), jnp.int32),
            pltpu.VMEM((128,), jnp.int32),
        ],
        compiler_params=pltpu.CompilerParams(needs_layout_passes=False),
    )
    def _sc_route(e_hbm, w_hbm, p01_hbm, ws_hbm,
                  e_v, w_v, rank_v, p01_v, ws_v, cnt_v, bs_v, cb_v):
        wid = lax.axis_index("s") * 2 + lax.axis_index("c")

        @pl.when(wid == 0)
        def _():
            pltpu.sync_copy(e_hbm, e_v)
            pltpu.sync_copy(w_hbm, w_v)
            cnt_v[pl.ds(0, 16)] = jnp.zeros((16,), jnp.int32)

            def p1(i, carry):
                sl = pl.ds(i * 16, 16)
                ids = e_v[sl]
                baseg = plsc.load_gather(cnt_v, [ids])
                occ, lastm = plsc.scan_count(ids)
                rank = baseg + occ - 1
                rank_v[sl] = rank
                plsc.store_scatter(cnt_v, [ids], rank + 1, mask=lastm)
                return carry

            lax.fori_loop(0, A // 16, p1, 0)

            cnts = cnt_v[pl.ds(0, 16)]
            bpe = (cnts + BLK - 1) // BLK
            cbv = plsc.cumsum(bpe)
            bs_v[pl.ds(0, 16)] = (cbv - bpe) * BLK

            def p2(i, carry):
                sl = pl.ds(i * 16, 16)
                ids = e_v[sl]
                bsg = plsc.load_gather(bs_v, [ids])
                pos = bsg + rank_v[sl]
                plsc.store_scatter(ws_v, [pos], w_v[sl])
                a_v = lax.broadcasted_iota(jnp.int32, (16,), 0) + i * 16
                dest = (a_v & (-128)) + ((a_v & 1) << 6) + ((a_v & 127) >> 1)
                plsc.store_scatter(p01_v, [dest], pos)
                return carry

            lax.fori_loop(0, A // 16, p2, 0)
            pltpu.sync_copy(p01_v, p01_hbm)
            pltpu.sync_copy(ws_v, ws_hbm)

    return _sc_route


@functools.cache
def _sc_dispatch_call():
    @functools.partial(
        pl.kernel,
        out_type=jax.ShapeDtypeStruct((APAD, H), jnp.float32),
        mesh=_sc_mesh(),
        scratch_types=[
            pltpu.VMEM((K, CROWS), jnp.int32),
            pltpu.VMEM((CROWS, H), jnp.float32),
            pltpu.SemaphoreType.DMA,
        ],
    )
    def _sc_dispatch(p01_hbm, hid_hbm, out_hbm, idx_v, rows_v, ssem):
        wid = lax.axis_index("s") * 2 + lax.axis_index("c")
        base = wid * CROWS
        pltpu.sync_copy(p01_hbm.at[wid], idx_v)
        pltpu.sync_copy(hid_hbm.at[pl.ds(base, CROWS)], rows_v)
        s0 = pltpu.async_copy(rows_v, out_hbm.at[idx_v.at[0]], ssem)
        s1 = pltpu.async_copy(rows_v, out_hbm.at[idx_v.at[1]], ssem)
        s0.wait()
        s1.wait()

    return _sc_dispatch


@functools.cache
def _sc_combine_call():
    @functools.partial(
        pl.kernel,
        out_type=jax.ShapeDtypeStruct((T, H), jnp.float32),
        mesh=_sc_mesh(),
        scratch_types=[
            pltpu.VMEM((K * CROWS,), jnp.int32),
            pltpu.VMEM((CCH, H), jnp.float32),
            pltpu.VMEM((CCH, H), jnp.float32),
            pltpu.SemaphoreType.DMA,
        ],
    )
    def _sc_combine(p01_hbm, y_hbm, out_hbm, idx_v, r0_v, r1_v, sem):
        wid = lax.axis_index("s") * 2 + lax.axis_index("c")
        pltpu.sync_copy(p01_hbm.at[wid], idx_v)
        for c in range(CROWS // CCH):
            base = wid * CROWS + c * CCH
            cp0 = pltpu.async_copy(y_hbm.at[idx_v.at[pl.ds(c * CCH, CCH)]],
                                   r0_v, sem)
            cp1 = pltpu.async_copy(y_hbm.at[idx_v.at[pl.ds(CROWS + c * CCH, CCH)]],
                                   r1_v, sem)
            cp0.wait()
            cp1.wait()

            def _add16(i, _):
                r = i // (H // 16)
                k = (i % (H // 16)) * 16
                r0_v[r, pl.ds(k, 16)] = r0_v[r, pl.ds(k, 16)] + r1_v[r, pl.ds(k, 16)]
                return 0

            lax.fori_loop(0, CCH * (H // 16), _add16, 0)
            pltpu.sync_copy(r0_v, out_hbm.at[pl.ds(base, CCH)])

    return _sc_combine


def _ffn_body(be_ref, x_ref, g_ref, u_ref, d_ref, w_ref, o_ref):
    j = pl.program_id(0)
    b = pl.program_id(1)

    @pl.when(be_ref[b] < E)
    def _():
        x = x_ref[...]
        g = lax.dot_general(x, g_ref[0], (((1,), (1,)), ((), ())),
                            preferred_element_type=jnp.float32)
        u = lax.dot_general(x, u_ref[0], (((1,), (1,)), ((), ())),
                            preferred_element_type=jnp.float32)
        a = jax.nn.gelu(g, approximate=True) * u
        a = a * w_ref[0, 0, :][:, None]
        p = lax.dot_general(a, d_ref[0], (((1,), (1,)), ((), ())),
                            preferred_element_type=jnp.float32)

        @pl.when(j == 0)
        def _():
            o_ref[pl.ds(b * BLK, BLK), :] = p

        @pl.when(j != 0)
        def _():
            o_ref[pl.ds(b * BLK, BLK), :] += p


_FFN_GRID = pltpu.PrefetchScalarGridSpec(
    num_scalar_prefetch=1,
    grid=(NJ, NB),
    in_specs=[
        pl.BlockSpec((BLK, H),
                     lambda j, b, be: (jnp.where(be[b] < E, b, 0), 0)),
        pl.BlockSpec((1, CH, H),
                     lambda j, b, be: (jnp.minimum(be[b], E - 1), j, 0)),
        pl.BlockSpec((1, CH, H),
                     lambda j, b, be: (jnp.minimum(be[b], E - 1), NJ + j, 0)),
        pl.BlockSpec((1, H, CH),
                     lambda j, b, be: (jnp.minimum(be[b], E - 1), 0, j)),
        pl.BlockSpec((1, 1, BLK), lambda j, b, be: (b, 0, 0)),
    ],
    out_specs=pl.BlockSpec((APAD, H), lambda j, b, be: (0, 0)),
)

_ffn_call = pl.pallas_call(
    _ffn_body,
    grid_spec=_FFN_GRID,
    out_shape=jax.ShapeDtypeStruct((APAD, H), jnp.float32),
    compiler_params=pltpu.CompilerParams(
        dimension_semantics=("arbitrary", "arbitrary"),
    ),
)


def kernel(hidden_states, top_k_index, top_k_weights, gate_up_proj, down_proj):
    e_flat = top_k_index.reshape(A).astype(jnp.int32)
    w_flat = top_k_weights.reshape(A).astype(jnp.float32)

    p01_flat, row_weight = _sc_route_call()(e_flat, w_flat)
    counts = jnp.sum(
        (e_flat[:, None] == jnp.arange(E, dtype=jnp.int32)[None, :])
        .astype(jnp.int32), axis=0)
    cb = jnp.cumsum((counts + BLK - 1) // BLK)
    block_expert = jnp.sum(
        (jnp.arange(NB, dtype=jnp.int32)[:, None] >= cb[None, :])
        .astype(jnp.int32), axis=1)
    x_sorted = _sc_dispatch_call()(p01_flat.reshape(NW, K, CROWS), hidden_states)
    y_sorted = _ffn_call(
        block_expert,
        x_sorted,
        gate_up_proj,
        gate_up_proj,
        down_proj,
        row_weight.reshape(NB, 1, BLK),
    )
    final = _sc_combine_call()(p01_flat.reshape(NW, K * CROWS), y_sorted)
    return final


# speedup vs baseline: 2.0454x; 1.0546x over previous
import functools

import jax
import jax.numpy as jnp
from jax import lax
from jax.experimental import pallas as pl
from jax.experimental.pallas import tpu as pltpu
from jax.experimental.pallas import tpu_sc as plsc

E = 8
K = 2
T = 2048
H = 1024
F = 4096
A = T * K
BLK = 256
NB = 24
APAD = NB * BLK
NJ = 4
CH = F // NJ

NW = 32

GROWS = APAD // NW
GCH = 32
GNCH = GROWS // GCH
GNBUF = 3
CROWS = T // NW
CCH = 16


@functools.cache
def _sc_mesh():
    return plsc.VectorSubcoreMesh(core_axis_name="c", subcore_axis_name="s")


@functools.cache
def _sc_route_call():
    @functools.partial(
        pl.kernel,
        out_type=(
            jax.ShapeDtypeStruct((A,), jnp.int32),
            jax.ShapeDtypeStruct((APAD,), jnp.float32),
        ),
        mesh=_sc_mesh(),
        scratch_types=[
            pltpu.VMEM((A,), jnp.int32),
            pltpu.VMEM((A,), jnp.float32),
            pltpu.VMEM((A,), jnp.int32),
            pltpu.VMEM((A,), jnp.int32),
            pltpu.VMEM((APAD,), jnp.float32),
            pltpu.VMEM((128,), jnp.int32),
            pltpu.VMEM((128,), jnp.int32),
            pltpu.VMEM((128,), jnp.int32),
        ],
        compiler_params=pltpu.CompilerParams(needs_layout_passes=False),
    )
    def _sc_route(e_hbm, w_hbm, p01_hbm, ws_hbm,
                  e_v, w_v, rank_v, p01_v, ws_v, cnt_v, bs_v, cb_v):
        wid = lax.axis_index("s") * 2 + lax.axis_index("c")

        @pl.when(wid == 0)
        def _():
            pltpu.sync_copy(e_hbm, e_v)
            pltpu.sync_copy(w_hbm, w_v)
            cnt_v[pl.ds(0, 16)] = jnp.zeros((16,), jnp.int32)

            def p1(i, carry):
                sl = pl.ds(i * 16, 16)
                ids = e_v[sl]
                baseg = plsc.load_gather(cnt_v, [ids])
                occ, lastm = plsc.scan_count(ids)
                rank = baseg + occ - 1
                rank_v[sl] = rank
                plsc.store_scatter(cnt_v, [ids], rank + 1, mask=lastm)
                return carry

            lax.fori_loop(0, A // 16, p1, 0)

            cnts = cnt_v[pl.ds(0, 16)]
            bpe = (cnts + BLK - 1) // BLK
            cbv = plsc.cumsum(bpe)
            bs_v[pl.ds(0, 16)] = (cbv - bpe) * BLK

            def p2(i, carry):
                sl = pl.ds(i * 16, 16)
                ids = e_v[sl]
                bsg = plsc.load_gather(bs_v, [ids])
                pos = bsg + rank_v[sl]
                plsc.store_scatter(ws_v, [pos], w_v[sl])
                a_v = lax.broadcasted_iota(jnp.int32, (16,), 0) + i * 16
                dest = (a_v & (-128)) + ((a_v & 1) << 6) + ((a_v & 127) >> 1)
                plsc.store_scatter(p01_v, [dest], pos)
                return carry

            lax.fori_loop(0, A // 16, p2, 0)
            pltpu.sync_copy(p01_v, p01_hbm)
            pltpu.sync_copy(ws_v, ws_hbm)

    return _sc_route


@functools.cache
def _sc_dispatch_call():
    @functools.partial(
        pl.kernel,
        out_type=jax.ShapeDtypeStruct((APAD, H), jnp.float32),
        mesh=_sc_mesh(),
        scratch_types=[
            pltpu.VMEM((K, CROWS), jnp.int32),
            pltpu.VMEM((CROWS, H), jnp.float32),
            pltpu.SemaphoreType.DMA,
        ],
    )
    def _sc_dispatch(p01_hbm, hid_hbm, out_hbm, idx_v, rows_v, ssem):
        wid = lax.axis_index("s") * 2 + lax.axis_index("c")
        base = wid * CROWS
        pltpu.sync_copy(p01_hbm.at[wid], idx_v)
        pltpu.sync_copy(hid_hbm.at[pl.ds(base, CROWS)], rows_v)
        s0 = pltpu.async_copy(rows_v, out_hbm.at[idx_v.at[0]], ssem)
        s1 = pltpu.async_copy(rows_v, out_hbm.at[idx_v.at[1]], ssem)
        s0.wait()
        s1.wait()

    return _sc_dispatch


@functools.cache
def _sc_combine_call():
    @functools.partial(
        pl.kernel,
        out_type=jax.ShapeDtypeStruct((T, H), jnp.float32),
        mesh=_sc_mesh(),
        scratch_types=[
            pltpu.VMEM((K * CROWS,), jnp.int32),
            pltpu.VMEM((CCH, H), jnp.float32),
            pltpu.VMEM((CCH, H), jnp.float32),
            pltpu.VMEM((CCH, H), jnp.float32),
            pltpu.VMEM((CCH, H), jnp.float32),
            pltpu.SemaphoreType.DMA,
            pltpu.SemaphoreType.DMA,
        ],
    )
    def _sc_combine(p01_hbm, y_hbm, out_hbm, idx_v, r0a, r1a, r0b, r1b,
                    gsem, wsem):
        wid = lax.axis_index("s") * 2 + lax.axis_index("c")
        pltpu.sync_copy(p01_hbm.at[wid], idx_v)
        pairs = ((r0a, r1a), (r0b, r1b))
        nch = CROWS // CCH

        def fire(c):
            pair = pairs[c % 2]
            return (
                pltpu.async_copy(
                    y_hbm.at[idx_v.at[pl.ds(c * CCH, CCH)]], pair[0], gsem),
                pltpu.async_copy(
                    y_hbm.at[idx_v.at[pl.ds(CROWS + c * CCH, CCH)]], pair[1],
                    gsem),
            )

        g = [None] * nch
        wb = [None] * nch
        g[0] = fire(0)
        for c in range(nch):
            if c + 1 < nch:
                if c >= 1:
                    wb[c - 1].wait()
                g[c + 1] = fire(c + 1)
            g[c][0].wait()
            g[c][1].wait()
            pair = pairs[c % 2]

            def _add16(i, _, pair=pair):
                for u in range(4):
                    ii = i * 4 + u
                    r = ii // (H // 16)
                    k = (ii % (H // 16)) * 16
                    pair[0][r, pl.ds(k, 16)] = (
                        pair[0][r, pl.ds(k, 16)] + pair[1][r, pl.ds(k, 16)])
                return 0

            lax.fori_loop(0, CCH * (H // 16) // 4, _add16, 0)
            wb[c] = pltpu.async_copy(
                pair[0], out_hbm.at[pl.ds(wid * CROWS + c * CCH, CCH)], wsem)
        wb[nch - 2].wait()
        wb[nch - 1].wait()

    return _sc_combine


def _ffn_body(be_ref, x_ref, g_ref, u_ref, d_ref, w_ref, o_ref):
    j = pl.program_id(0)
    b = pl.program_id(1)

    @pl.when(be_ref[b] < E)
    def _():
        x = x_ref[...]
        g = lax.dot_general(x, g_ref[0], (((1,), (1,)), ((), ())),
                            preferred_element_type=jnp.float32)
        u = lax.dot_general(x, u_ref[0], (((1,), (1,)), ((), ())),
                            preferred_element_type=jnp.float32)
        a = jax.nn.gelu(g, approximate=True) * u
        a = a * w_ref[0, 0, :][:, None]
        p = lax.dot_general(a, d_ref[0], (((1,), (1,)), ((), ())),
                            preferred_element_type=jnp.float32)

        @pl.when(j == 0)
        def _():
            o_ref[pl.ds(b * BLK, BLK), :] = p

        @pl.when(j != 0)
        def _():
            o_ref[pl.ds(b * BLK, BLK), :] += p


_FFN_GRID = pltpu.PrefetchScalarGridSpec(
    num_scalar_prefetch=1,
    grid=(NJ, NB),
    in_specs=[
        pl.BlockSpec((BLK, H),
                     lambda j, b, be: (jnp.where(be[b] < E, b, 0), 0)),
        pl.BlockSpec((1, CH, H),
                     lambda j, b, be: (jnp.minimum(be[b], E - 1), j, 0)),
        pl.BlockSpec((1, CH, H),
                     lambda j, b, be: (jnp.minimum(be[b], E - 1), NJ + j, 0)),
        pl.BlockSpec((1, H, CH),
                     lambda j, b, be: (jnp.minimum(be[b], E - 1), 0, j)),
        pl.BlockSpec((1, 1, BLK), lambda j, b, be: (b, 0, 0)),
    ],
    out_specs=pl.BlockSpec((APAD, H), lambda j, b, be: (0, 0)),
)

_ffn_call = pl.pallas_call(
    _ffn_body,
    grid_spec=_FFN_GRID,
    out_shape=jax.ShapeDtypeStruct((APAD, H), jnp.float32),
    compiler_params=pltpu.CompilerParams(
        dimension_semantics=("arbitrary", "arbitrary"),
    ),
)


def kernel(hidden_states, top_k_index, top_k_weights, gate_up_proj, down_proj):
    e_flat = top_k_index.reshape(A).astype(jnp.int32)
    w_flat = top_k_weights.reshape(A).astype(jnp.float32)

    p01_flat, row_weight = _sc_route_call()(e_flat, w_flat)
    counts = jnp.sum(
        (e_flat[:, None] == jnp.arange(E, dtype=jnp.int32)[None, :])
        .astype(jnp.int32), axis=0)
    cb = jnp.cumsum((counts + BLK - 1) // BLK)
    block_expert = jnp.sum(
        (jnp.arange(NB, dtype=jnp.int32)[:, None] >= cb[None, :])
        .astype(jnp.int32), axis=1)
    x_sorted = _sc_dispatch_call()(p01_flat.reshape(NW, K, CROWS), hidden_states)
    y_sorted = _ffn_call(
        block_expert,
        x_sorted,
        gate_up_proj,
        gate_up_proj,
        down_proj,
        row_weight.reshape(NB, 1, BLK),
    )
    final = _sc_combine_call()(p01_flat.reshape(NW, K * CROWS), y_sorted)
    return final
